# baseline (speedup 1.0000x reference)
import functools

import jax
import jax.numpy as jnp
from jax.experimental import pallas as pl

_ROW_BLK = 2048


def _lin_body(x_ref, w_ref, b_ref, o_ref, *, act):
    y = jnp.dot(x_ref[...], w_ref[...], preferred_element_type=jnp.float32)
    y = y + b_ref[...]
    if act:
        y = jnp.maximum(y, 0.0)
    o_ref[...] = y


def _linear(x, w, b, act=False, blk=_ROW_BLK):
    n, k = x.shape
    m = w.shape[1]
    npad = (-n) % blk
    if npad:
        x = jnp.pad(x, ((0, npad), (0, 0)))
    ntot = x.shape[0]
    out = pl.pallas_call(
        functools.partial(_lin_body, act=act),
        grid=(ntot // blk,),
        in_specs=[
            pl.BlockSpec((blk, k), lambda i: (i, 0)),
            pl.BlockSpec((k, m), lambda i: (0, 0)),
            pl.BlockSpec((1, m), lambda i: (0, 0)),
        ],
        out_specs=pl.BlockSpec((blk, m), lambda i: (i, 0)),
        out_shape=jax.ShapeDtypeStruct((ntot, m), jnp.float32),
    )(x, w, b.reshape(1, m))
    return out[:n]


def _sage_lin_body(mean_ref, x_ref, wl_ref, wr_ref, b_ref, o_ref):
    y = jnp.dot(mean_ref[...], wl_ref[...], preferred_element_type=jnp.float32)
    y = y + jnp.dot(x_ref[...], wr_ref[...], preferred_element_type=jnp.float32)
    o_ref[...] = jnp.maximum(y + b_ref[...], 0.0)


def _sage_linear(mean, x, wl, wr, b, blk=_ROW_BLK):
    n, k = x.shape
    m = wl.shape[1]
    npad = (-n) % blk
    if npad:
        mean = jnp.pad(mean, ((0, npad), (0, 0)))
        x = jnp.pad(x, ((0, npad), (0, 0)))
    ntot = x.shape[0]
    out = pl.pallas_call(
        _sage_lin_body,
        grid=(ntot // blk,),
        in_specs=[
            pl.BlockSpec((blk, k), lambda i: (i, 0)),
            pl.BlockSpec((blk, k), lambda i: (i, 0)),
            pl.BlockSpec((k, m), lambda i: (0, 0)),
            pl.BlockSpec((k, m), lambda i: (0, 0)),
            pl.BlockSpec((1, m), lambda i: (0, 0)),
        ],
        out_specs=pl.BlockSpec((blk, m), lambda i: (i, 0)),
        out_shape=jax.ShapeDtypeStruct((ntot, m), jnp.float32),
    )(mean, x, wl, wr, b.reshape(1, m))
    return out[:n]


def _attn_body(hi_ref, hj_ref, al_ref, ar_ref, o_ref):
    hi = hi_ref[...]
    hj = hj_ref[...]
    logits = jnp.sum(hi * hj, axis=-1, keepdims=True)
    alpha = (jnp.sum(hi * al_ref[...], axis=-1, keepdims=True)
             + jnp.sum(hj * ar_ref[...], axis=-1, keepdims=True))
    alpha = alpha * jax.nn.sigmoid(logits)
    o_ref[...] = jnp.where(alpha >= 0.0, alpha, 0.2 * alpha)


def _attn_scores(hi, hj, al, ar, blk=_ROW_BLK):
    e, d = hi.shape
    epad = (-e) % blk
    if epad:
        hi = jnp.pad(hi, ((0, epad), (0, 0)))
        hj = jnp.pad(hj, ((0, epad), (0, 0)))
    etot = hi.shape[0]
    out = pl.pallas_call(
        _attn_body,
        grid=(etot // blk,),
        in_specs=[
            pl.BlockSpec((blk, d), lambda i: (i, 0)),
            pl.BlockSpec((blk, d), lambda i: (i, 0)),
            pl.BlockSpec((1, d), lambda i: (0, 0)),
            pl.BlockSpec((1, d), lambda i: (0, 0)),
        ],
        out_specs=pl.BlockSpec((blk, 1), lambda i: (i, 0)),
        out_shape=jax.ShapeDtypeStruct((etot, 1), jnp.float32),
    )(hi, hj, al.reshape(1, d), ar.reshape(1, d))
    return out[:e, 0]


def _sage_conv(x, ei, wl, b, wr):
    src, dst = ei[0], ei[1]
    n = x.shape[0]
    agg = jax.ops.segment_sum(x[src], dst, num_segments=n)
    deg = jax.ops.segment_sum(jnp.ones((src.shape[0],), jnp.float32), dst,
                              num_segments=n)
    mean = agg / jnp.maximum(deg, 1.0)[:, None]
    return _sage_linear(mean, x, wl, wr, b)


def _supergat_conv(x, ei, w, al, ar, b):
    src, dst = ei[0], ei[1]
    n = x.shape[0]
    h = _linear(x, w, jnp.zeros((w.shape[1],), jnp.float32))
    hj = h[src]
    hi = h[dst]
    alpha = _attn_scores(hi, hj, al, ar)
    amax = jax.ops.segment_max(alpha, dst, num_segments=n)
    amax = jnp.where(jnp.isfinite(amax), amax, 0.0)
    ex = jnp.exp(alpha - amax[dst])
    denom = jax.ops.segment_sum(ex, dst, num_segments=n)
    coef = ex / jnp.maximum(denom[dst], 1e-16)
    out = jax.ops.segment_sum(hj * coef[:, None], dst, num_segments=n)
    return jnp.maximum(out + b, 0.0)


def _gep(x, batch, num_graphs):
    s = jax.ops.segment_sum(x, batch, num_segments=num_graphs)
    c = jax.ops.segment_sum(jnp.ones((x.shape[0],), jnp.float32), batch,
                            num_segments=num_graphs)
    return s / jnp.maximum(c, 1.0)[:, None]


def _branch(x, ei, batch, c1_Wl, c1_Wr, c1_b, c2_Wl, c2_Wr, c2_b,
            c3_W, c3_al, c3_ar, c3_b, fc1_W, fc1_b, fc2_W, fc2_b):
    x = _sage_conv(x, ei, c1_Wl, c1_b, c1_Wr)
    x = _sage_conv(x, ei, c2_Wl, c2_b, c2_Wr)
    x = _supergat_conv(x, ei, c3_W, c3_al, c3_ar, c3_b)
    x = _gep(x, batch, 256)
    x = _linear(x, fc1_W, fc1_b, act=True, blk=256)
    x = _linear(x, fc2_W, fc2_b, act=False, blk=256)
    return x


def kernel(lig_x, pro_x, lig_edge_index, pro_edge_index, lig_batch, pro_batch,
           lc1_Wl, lc1_Wr, lc1_b, lc2_Wl, lc2_Wr, lc2_b,
           lc3_W, lc3_al, lc3_ar, lc3_b,
           lfc1_W, lfc1_b, lfc2_W, lfc2_b,
           pc1_Wl, pc1_Wr, pc1_b, pc2_Wl, pc2_Wr, pc2_b,
           pc3_W, pc3_al, pc3_ar, pc3_b,
           pfc1_W, pfc1_b, pfc2_W, pfc2_b,
           fc1_W, fc1_b, fc2_W, fc2_b, out_W, out_b):
    x = _branch(lig_x, lig_edge_index, lig_batch,
                lc1_Wl, lc1_Wr, lc1_b, lc2_Wl, lc2_Wr, lc2_b,
                lc3_W, lc3_al, lc3_ar, lc3_b,
                lfc1_W, lfc1_b, lfc2_W, lfc2_b)
    xt = _branch(pro_x, pro_edge_index, pro_batch,
                 pc1_Wl, pc1_Wr, pc1_b, pc2_Wl, pc2_Wr, pc2_b,
                 pc3_W, pc3_al, pc3_ar, pc3_b,
                 pfc1_W, pfc1_b, pfc2_W, pfc2_b)
    xc = jnp.concatenate([x, xt], axis=1)
    xc = _linear(xc, fc1_W, fc1_b, act=True, blk=256)
    xc = _linear(xc, fc2_W, fc2_b, act=True, blk=256)
    return _linear(xc, out_W, out_b, act=False, blk=256)
